# baseline (speedup 1.0000x reference)
import jax
import jax.numpy as jnp
from jax.experimental import pallas as pl
from jax.experimental.pallas import tpu as pltpu


def _identity_copy_kernel(x_ref, o_ref):
    o_ref[...] = x_ref[...]


def kernel(x, codebook):
    del codebook
    x2 = x.reshape(16 * 576, 256)
    out = pl.pallas_call(
        _identity_copy_kernel,
        grid=(2,),
        in_specs=[pl.BlockSpec((4608, 256), lambda i: (i, 0))],
        out_specs=pl.BlockSpec((4608, 256), lambda i: (i, 0)),
        out_shape=jax.ShapeDtypeStruct((16 * 576, 256), x.dtype),
        compiler_params=pltpu.CompilerParams(
            dimension_semantics=("arbitrary",),
        ),
    )(x2)
    return out.reshape(x.shape)
